# baseline (speedup 1.0000x reference)
import functools

import jax
import jax.numpy as jnp
from jax import lax
from jax.experimental import pallas as pl
from jax.experimental.pallas import tpu as pltpu
from jax.experimental.pallas import tpu_sc as plsc

N_NODES = 10000
N_EDGES = 160000
HID = 512
D_INV_SQRT = 1.0 / (512.0 ** 0.5)
EPS = 1e-16

BM = 400

NP = 10240
RTILE = 80
PASSES = NP // (32 * RTILE)
STRIP = 1600
MBUF = 1600
G = 16
NSTRIPS = N_EDGES // STRIP


def _mm_bias_body(x_ref, w_ref, b_ref, o_ref):
    o_ref[...] = (
        jnp.dot(x_ref[...], w_ref[...], preferred_element_type=jnp.float32)
        + b_ref[...]
    )


def _mm_bias(x, w, b, bm=BM):
    m, kdim = x.shape
    n = w.shape[1]
    return pl.pallas_call(
        _mm_bias_body,
        grid=(m // bm,),
        in_specs=[
            pl.BlockSpec((bm, kdim), lambda i: (i, 0)),
            pl.BlockSpec((kdim, n), lambda i: (0, 0)),
            pl.BlockSpec((1, n), lambda i: (0, 0)),
        ],
        out_specs=pl.BlockSpec((bm, n), lambda i: (i, 0)),
        out_shape=jax.ShapeDtypeStruct((m, n), jnp.float32),
    )(x, w, b.reshape(1, n))


def _gate_body(raw_ref, r_ref, d_ref, s2_ref, we_ref, ac_ref, bc_ref, o_ref):
    inv_d = 1.0 / (d_ref[...] + EPS)
    out = (raw_ref[...] + s2_ref[...] * we_ref[...]) * inv_d
    r = r_ref[...]
    bl = jnp.sum(out * ac_ref[...] + r * bc_ref[...], axis=1, keepdims=True)
    beta = jax.nn.sigmoid(bl)
    o_ref[...] = jnp.maximum(beta * r + (1.0 - beta) * out, 0.0)


def _gate(out_raw, r, denom, s2, we_row, w_ac, w_bc):
    n = out_raw.shape[0]
    vspec = pl.BlockSpec((1, HID), lambda i: (0, 0))
    return pl.pallas_call(
        _gate_body,
        grid=(n // BM,),
        in_specs=[
            pl.BlockSpec((BM, HID), lambda i: (i, 0)),
            pl.BlockSpec((BM, HID), lambda i: (i, 0)),
            pl.BlockSpec((BM, 1), lambda i: (i, 0)),
            pl.BlockSpec((BM, 1), lambda i: (i, 0)),
            vspec, vspec, vspec,
        ],
        out_specs=pl.BlockSpec((BM, HID), lambda i: (i, 0)),
        out_shape=jax.ShapeDtypeStruct((n, HID), jnp.float32),
    )(out_raw, r, denom.reshape(n, 1), s2.reshape(n, 1),
      we_row.reshape(1, HID), w_ac.reshape(1, HID), w_bc.reshape(1, HID))


def _mlp_body(f_ref, w1_ref, b1_ref, g_ref, be_ref, w2_ref, b2_ref,
              wh_ref, bh_ref, o_ref):
    h = jnp.dot(f_ref[...], w1_ref[...], preferred_element_type=jnp.float32)
    h = jnp.maximum(h + b1_ref[...], 0.0)
    mu = jnp.mean(h, axis=1, keepdims=True)
    var = jnp.mean((h - mu) ** 2, axis=1, keepdims=True)
    h = (h - mu) * jax.lax.rsqrt(var + 1e-5) * g_ref[...] + be_ref[...]
    h = jnp.dot(h, w2_ref[...], preferred_element_type=jnp.float32)
    h = jnp.maximum(h + b2_ref[...], 0.0)
    o_ref[...] = (
        jnp.dot(h, wh_ref[...], preferred_element_type=jnp.float32)
        + bh_ref[...]
    )


def _mlp(feats, mp):
    b = feats.shape[0]
    bm = 512
    d1 = mp["W1"].shape[0]
    d2 = mp["W1"].shape[1]
    d3 = mp["W2"].shape[1]
    return pl.pallas_call(
        _mlp_body,
        grid=(b // bm,),
        in_specs=[
            pl.BlockSpec((bm, d1), lambda i: (i, 0)),
            pl.BlockSpec((d1, d2), lambda i: (0, 0)),
            pl.BlockSpec((1, d2), lambda i: (0, 0)),
            pl.BlockSpec((1, d2), lambda i: (0, 0)),
            pl.BlockSpec((1, d2), lambda i: (0, 0)),
            pl.BlockSpec((d2, d3), lambda i: (0, 0)),
            pl.BlockSpec((1, d3), lambda i: (0, 0)),
            pl.BlockSpec((d3, 1), lambda i: (0, 0)),
            pl.BlockSpec((1, 1), lambda i: (0, 0)),
        ],
        out_specs=pl.BlockSpec((bm, 1), lambda i: (i, 0)),
        out_shape=jax.ShapeDtypeStruct((b, 1), jnp.float32),
    )(feats, mp["W1"], mp["b1"].reshape(1, d2), mp["ln_g"].reshape(1, d2),
      mp["ln_b"].reshape(1, d2), mp["W2"], mp["b2"].reshape(1, d3),
      mp["Wh"], mp["bh"].reshape(1, 1))


def _edge_kernel(q_h, k_h, v_h, we_h, src_h, dst_h, ea_h,
                 out_h, den_h, s2_h,
                 we_v, dst_s, src_s, ea_s, srcm, dlm, eam,
                 qb, kb, vb, srcb, dgb, dlb, wbuf, weab,
                 acc_t, dacc_t, sacc_t, sem):
    c = lax.axis_index("c")
    s = lax.axis_index("s")
    wid = s * 2 + c
    z16f = jnp.zeros((16,), jnp.float32)
    lane = lax.iota(jnp.int32, 16)
    pltpu.sync_copy(we_h, we_v)

    def do_pass(p, carry):
        lo = p * (32 * RTILE) + wid * RTILE

        def zrow(rr, carry2):
            for i in range(HID // 16):
                acc_t[rr, pl.ds(16 * i, 16)] = z16f
            dacc_t[rr, :] = z16f
            sacc_t[rr, :] = z16f
            return carry2

        lax.fori_loop(0, RTILE, zrow, 0)

        def do_strip(st, carry2):
            base = st * STRIP
            pltpu.sync_copy(dst_h.at[pl.ds(base, STRIP)], dst_s)
            pltpu.sync_copy(src_h.at[pl.ds(base, STRIP)], src_s)
            pltpu.sync_copy(ea_h.at[pl.ds(base, STRIP)], ea_s)

            def filt(t, off):
                d16 = dst_s[pl.ds(16 * t, 16)]
                m = (d16 >= lo) & (d16 < lo + RTILE)
                plsc.store_compressed(srcm.at[pl.ds(off, 16)],
                                      src_s[pl.ds(16 * t, 16)], mask=m)
                plsc.store_compressed(dlm.at[pl.ds(off, 16)], d16 - lo,
                                      mask=m)
                plsc.store_compressed(eam.at[pl.ds(off, 16)],
                                      ea_s[pl.ds(16 * t, 16)], mask=m)
                return off + plsc.all_reduce_population_count(m)[0]

            mcount = lax.fori_loop(0, STRIP // 16, filt, 0)
            nblk = (mcount + G - 1) // G

            def blk(g, carry3):
                gb = g * G
                for t in range(G // 16):
                    pos = gb + 16 * t + lane
                    mm = pos < mcount
                    sv = jnp.where(mm, srcm[pl.ds(gb + 16 * t, 16)], 0)
                    dv = jnp.where(mm, dlm[pl.ds(gb + 16 * t, 16)], 0)
                    srcb[pl.ds(16 * t, 16)] = sv
                    dlb[pl.ds(16 * t, 16)] = dv
                    dgb[pl.ds(16 * t, 16)] = jnp.where(mm, dv + lo, 0)
                cq = pltpu.async_copy(q_h.at[dgb], qb, sem)
                ck = pltpu.async_copy(k_h.at[srcb], kb, sem)
                cv = pltpu.async_copy(v_h.at[srcb], vb, sem)
                cq.wait()
                ck.wait()
                cv.wait()

                def dot16(t16, carry4):
                    av = eam[pl.ds(gb + 16 * t16, 16)]
                    lvec = z16f
                    for t in range(16):
                        e = t16 * 16 + t
                        ea_e = av[t]
                        a = z16f
                        for i in range(HID // 16):
                            kk = (kb[e, pl.ds(16 * i, 16)]
                                  + ea_e * we_v[pl.ds(16 * i, 16)])
                            a = a + qb[e, pl.ds(16 * i, 16)] * kk
                        lvec = jnp.where(lane == t, jnp.sum(a), lvec)
                    pos = gb + 16 * t16 + lane
                    live = pos < mcount
                    w = jnp.where(live, jnp.exp(lvec * D_INV_SQRT), 0.0)
                    wbuf[pl.ds(16 * t16, 16)] = w
                    weab[pl.ds(16 * t16, 16)] = jnp.where(live, w * av, 0.0)
                    return carry4

                lax.fori_loop(0, G // 16, dot16, 0)

                def accrow16(t16, carry4):
                    wv = wbuf[pl.ds(16 * t16, 16)]
                    wev = weab[pl.ds(16 * t16, 16)]
                    dv = dlb[pl.ds(16 * t16, 16)]
                    for t in range(16):
                        e = t16 * 16 + t
                        ws = wv[t]
                        dl = dv[t]
                        for i in range(HID // 16):
                            plsc.addupdate(
                                acc_t.at[dl, pl.ds(16 * i, 16)],
                                vb[e, pl.ds(16 * i, 16)] * ws)
                        plsc.addupdate(dacc_t.at[dl], z16f + ws)
                        plsc.addupdate(sacc_t.at[dl], z16f + wev[t])
                    return carry4

                lax.fori_loop(0, G // 16, accrow16, 0)
                return carry3

            lax.fori_loop(0, nblk, blk, 0)
            return carry2

        lax.fori_loop(0, NSTRIPS, do_strip, 0)
        pltpu.sync_copy(acc_t, out_h.at[pl.ds(lo, RTILE)])
        pltpu.sync_copy(dacc_t, den_h.at[pl.ds(lo, RTILE)])
        pltpu.sync_copy(sacc_t, s2_h.at[pl.ds(lo, RTILE)])
        return carry

    lax.fori_loop(0, PASSES, do_pass, 0)


def _edge_phase(q, k, v, we_row, src, dst, ea):
    mesh = plsc.VectorSubcoreMesh(core_axis_name="c", subcore_axis_name="s")
    f32, i32 = jnp.float32, jnp.int32
    ek = functools.partial(
        pl.kernel,
        out_type=[
            jax.ShapeDtypeStruct((NP, HID), f32),
            jax.ShapeDtypeStruct((NP, 16), f32),
            jax.ShapeDtypeStruct((NP, 16), f32),
        ],
        mesh=mesh,
        compiler_params=pltpu.CompilerParams(needs_layout_passes=False),
        scratch_types=[
            pltpu.VMEM((HID,), f32),
            pltpu.VMEM((STRIP,), i32),
            pltpu.VMEM((STRIP,), i32),
            pltpu.VMEM((STRIP,), f32),
            pltpu.VMEM((MBUF,), i32),
            pltpu.VMEM((MBUF,), i32),
            pltpu.VMEM((MBUF,), f32),
            pltpu.VMEM((G, HID), f32),
            pltpu.VMEM((G, HID), f32),
            pltpu.VMEM((G, HID), f32),
            pltpu.VMEM((G,), i32),
            pltpu.VMEM((G,), i32),
            pltpu.VMEM((G,), i32),
            pltpu.VMEM((G,), f32),
            pltpu.VMEM((G,), f32),
            pltpu.VMEM((RTILE, HID), f32),
            pltpu.VMEM((RTILE, 16), f32),
            pltpu.VMEM((RTILE, 16), f32),
            pltpu.SemaphoreType.DMA,
        ],
    )(_edge_kernel)
    out_p, den_p, s2_p = ek(q, k, v, we_row, src, dst, ea)
    return out_p[:N_NODES], den_p[:N_NODES, 0], s2_p[:N_NODES, 0]


def _gather_kernel(t_h, i_h, o_h, idx_v, rows_v, sem):
    wid = lax.axis_index("s") * 2 + lax.axis_index("c")
    base = wid * 512
    for ch in range(4):
        off = base + ch * 128
        pltpu.sync_copy(i_h.at[pl.ds(off, 128)], idx_v)
        pltpu.async_copy(t_h.at[idx_v], rows_v, sem).wait()
        pltpu.sync_copy(rows_v, o_h.at[pl.ds(off, 128)])


def _gather_rows(table, idx):
    mesh = plsc.VectorSubcoreMesh(core_axis_name="c", subcore_axis_name="s")
    gk = functools.partial(
        pl.kernel,
        out_type=jax.ShapeDtypeStruct((idx.shape[0], HID), jnp.float32),
        mesh=mesh,
        compiler_params=pltpu.CompilerParams(needs_layout_passes=False),
        scratch_types=[
            pltpu.VMEM((128,), jnp.int32),
            pltpu.VMEM((128, HID), jnp.float32),
            pltpu.SemaphoreType.DMA,
        ],
    )(_gather_kernel)
    return gk(table, idx)


def _layer(h, src, dst, ea, p, pad_k):
    n = h.shape[0]
    din = h.shape[1]
    if pad_k:
        hp = jnp.pad(h, ((0, 0), (0, pad_k)))
        wcat = jnp.pad(
            jnp.concatenate([p["Wq"], p["Wk"], p["Wv"], p["Wskip"]], axis=1),
            ((0, pad_k), (0, 0)))
    else:
        hp = h
        wcat = jnp.concatenate([p["Wq"], p["Wk"], p["Wv"], p["Wskip"]], axis=1)
    bcat = jnp.concatenate([p["bq"], p["bk"], p["bv"], p["bskip"]])
    qkvr = _mm_bias(hp, wcat, bcat)
    q = qkvr[:, :HID]
    k = qkvr[:, HID:2 * HID]
    v = qkvr[:, 2 * HID:3 * HID]
    r = qkvr[:, 3 * HID:]

    we_row = p["We"][0]
    out_raw, denom, s2 = _edge_phase(q, k, v, we_row, src, dst, ea)

    wb = p["Wbeta"][:, 0]
    w_ac = wb[:HID] + wb[2 * HID:]
    w_bc = wb[HID:2 * HID] - wb[2 * HID:]
    return _gate(out_raw, r, denom, s2, we_row, w_ac, w_bc)


def kernel(x, edge_index, edge_attr, start_idx, end_idx, x_1, x_2, params):
    src = edge_index[0]
    dst = edge_index[1]
    ea = edge_attr[:, 0]

    h = _layer(x, src, dst, ea, params["conv0"], pad_k=26)
    h = _layer(h, src, dst, ea, params["conv1"], pad_k=0)
    h = _layer(h, src, dst, ea, params["conv2"], pad_k=0)

    idx = jnp.stack([start_idx, end_idx, x_1, x_2], axis=1).reshape(-1)
    rows = _gather_rows(h, idx.astype(jnp.int32))
    feats = rows.reshape(start_idx.shape[0], 4 * HID)
    return _mlp(feats, params["mlp"])


# speedup vs baseline: 1.0259x; 1.0259x over previous
import functools

import jax
import jax.numpy as jnp
from jax import lax
from jax.experimental import pallas as pl
from jax.experimental.pallas import tpu as pltpu
from jax.experimental.pallas import tpu_sc as plsc

N_NODES = 10000
N_EDGES = 160000
HID = 512
D_INV_SQRT = 1.0 / (512.0 ** 0.5)
EPS = 1e-16

BM = 400

NP = 10240
RTILE = 80
PASSES = NP // (32 * RTILE)
STRIP = 1600
MBUF = 1600
G = 16
NSTRIPS = N_EDGES // STRIP


def _mm_bias_body(x_ref, w_ref, b_ref, o_ref):
    o_ref[...] = (
        jnp.dot(x_ref[...], w_ref[...], preferred_element_type=jnp.float32)
        + b_ref[...]
    )


def _mm_bias(x, w, b, bm=BM):
    m, kdim = x.shape
    n = w.shape[1]
    return pl.pallas_call(
        _mm_bias_body,
        grid=(m // bm,),
        in_specs=[
            pl.BlockSpec((bm, kdim), lambda i: (i, 0)),
            pl.BlockSpec((kdim, n), lambda i: (0, 0)),
            pl.BlockSpec((1, n), lambda i: (0, 0)),
        ],
        out_specs=pl.BlockSpec((bm, n), lambda i: (i, 0)),
        out_shape=jax.ShapeDtypeStruct((m, n), jnp.float32),
    )(x, w, b.reshape(1, n))


def _gate_body(raw_ref, r_ref, d_ref, s2_ref, we_ref, ac_ref, bc_ref, o_ref):
    inv_d = 1.0 / (d_ref[...] + EPS)
    out = (raw_ref[...] + s2_ref[...] * we_ref[...]) * inv_d
    r = r_ref[...]
    bl = jnp.sum(out * ac_ref[...] + r * bc_ref[...], axis=1, keepdims=True)
    beta = jax.nn.sigmoid(bl)
    o_ref[...] = jnp.maximum(beta * r + (1.0 - beta) * out, 0.0)


def _gate(out_raw, r, denom, s2, we_row, w_ac, w_bc):
    n = out_raw.shape[0]
    vspec = pl.BlockSpec((1, HID), lambda i: (0, 0))
    return pl.pallas_call(
        _gate_body,
        grid=(n // BM,),
        in_specs=[
            pl.BlockSpec((BM, HID), lambda i: (i, 0)),
            pl.BlockSpec((BM, HID), lambda i: (i, 0)),
            pl.BlockSpec((BM, 1), lambda i: (i, 0)),
            pl.BlockSpec((BM, 1), lambda i: (i, 0)),
            vspec, vspec, vspec,
        ],
        out_specs=pl.BlockSpec((BM, HID), lambda i: (i, 0)),
        out_shape=jax.ShapeDtypeStruct((n, HID), jnp.float32),
    )(out_raw, r, denom.reshape(n, 1), s2.reshape(n, 1),
      we_row.reshape(1, HID), w_ac.reshape(1, HID), w_bc.reshape(1, HID))


def _mlp_body(f_ref, w1_ref, b1_ref, g_ref, be_ref, w2_ref, b2_ref,
              wh_ref, bh_ref, o_ref):
    h = jnp.dot(f_ref[...], w1_ref[...], preferred_element_type=jnp.float32)
    h = jnp.maximum(h + b1_ref[...], 0.0)
    mu = jnp.mean(h, axis=1, keepdims=True)
    var = jnp.mean((h - mu) ** 2, axis=1, keepdims=True)
    h = (h - mu) * jax.lax.rsqrt(var + 1e-5) * g_ref[...] + be_ref[...]
    h = jnp.dot(h, w2_ref[...], preferred_element_type=jnp.float32)
    h = jnp.maximum(h + b2_ref[...], 0.0)
    o_ref[...] = (
        jnp.dot(h, wh_ref[...], preferred_element_type=jnp.float32)
        + bh_ref[...]
    )


def _mlp(feats, mp):
    b = feats.shape[0]
    bm = 512
    d1 = mp["W1"].shape[0]
    d2 = mp["W1"].shape[1]
    d3 = mp["W2"].shape[1]
    return pl.pallas_call(
        _mlp_body,
        grid=(b // bm,),
        in_specs=[
            pl.BlockSpec((bm, d1), lambda i: (i, 0)),
            pl.BlockSpec((d1, d2), lambda i: (0, 0)),
            pl.BlockSpec((1, d2), lambda i: (0, 0)),
            pl.BlockSpec((1, d2), lambda i: (0, 0)),
            pl.BlockSpec((1, d2), lambda i: (0, 0)),
            pl.BlockSpec((d2, d3), lambda i: (0, 0)),
            pl.BlockSpec((1, d3), lambda i: (0, 0)),
            pl.BlockSpec((d3, 1), lambda i: (0, 0)),
            pl.BlockSpec((1, 1), lambda i: (0, 0)),
        ],
        out_specs=pl.BlockSpec((bm, 1), lambda i: (i, 0)),
        out_shape=jax.ShapeDtypeStruct((b, 1), jnp.float32),
    )(feats, mp["W1"], mp["b1"].reshape(1, d2), mp["ln_g"].reshape(1, d2),
      mp["ln_b"].reshape(1, d2), mp["W2"], mp["b2"].reshape(1, d3),
      mp["Wh"], mp["bh"].reshape(1, 1))


def _edge_kernel(qkv_h, we_h, pk_h,
                 out_h, den_h, s2_h,
                 we_v, pb0, pb1, srcm, dlm, eam, qkvi, qkvb,
                 acc_t, dacc_t, sacc_t, semA, semB, semG):
    c = lax.axis_index("c")
    s = lax.axis_index("s")
    wid = s * 2 + c
    z16f = jnp.zeros((16,), jnp.float32)
    lane = lax.iota(jnp.int32, 16)
    pltpu.sync_copy(we_h, we_v)

    def do_pass(p, carry):
        lo = p * (32 * RTILE) + wid * RTILE

        def zrow(rr, carry2):
            for i in range(HID // 16):
                acc_t[rr, pl.ds(16 * i, 16)] = z16f
            dacc_t[rr, :] = z16f
            sacc_t[rr, :] = z16f
            return carry2

        lax.fori_loop(0, RTILE, zrow, 0)

        def process(st, pb):
            def filt(t, off):
                d16 = pb[0, pl.ds(16 * t, 16)]
                m = (d16 >= lo) & (d16 < lo + RTILE)
                plsc.store_compressed(srcm.at[pl.ds(off, 16)],
                                      pb[1, pl.ds(16 * t, 16)], mask=m)
                plsc.store_compressed(dlm.at[pl.ds(off, 16)], d16 - lo,
                                      mask=m)
                plsc.store_compressed(
                    eam.at[pl.ds(off, 16)],
                    plsc.bitcast(pb[2, pl.ds(16 * t, 16)], jnp.float32),
                    mask=m)
                return off + plsc.all_reduce_population_count(m)[0]

            mcount = lax.fori_loop(0, STRIP // 16, filt, 0)
            nblk = (mcount + G - 1) // G

            def blk(g, carry3):
                gb = g * G
                pos = gb + lane
                live = pos < mcount
                sv = jnp.where(live, srcm[pl.ds(gb, 16)], 0)
                dv = jnp.where(live, dlm[pl.ds(gb, 16)], 0)
                av = eam[pl.ds(gb, 16)]
                qkvi[pl.ds(0, 16)] = jnp.where(live, dv + lo, 0)
                qkvi[pl.ds(16, 16)] = sv + N_NODES
                qkvi[pl.ds(32, 16)] = sv + 2 * N_NODES
                pltpu.async_copy(qkv_h.at[qkvi], qkvb, semG).wait()

                lvec = z16f
                for t in range(16):
                    ea_e = av[t]
                    a0 = z16f
                    a1 = z16f
                    a2 = z16f
                    a3 = z16f
                    for i4 in range(0, HID // 16, 4):
                        accs = []
                        for u in range(4):
                            off_i = 16 * (i4 + u)
                            kk = (qkvb[16 + t, pl.ds(off_i, 16)]
                                  + ea_e * we_v[pl.ds(off_i, 16)])
                            accs.append(qkvb[t, pl.ds(off_i, 16)] * kk)
                        a0 = a0 + accs[0]
                        a1 = a1 + accs[1]
                        a2 = a2 + accs[2]
                        a3 = a3 + accs[3]
                    a = (a0 + a1) + (a2 + a3)
                    lvec = jnp.where(lane == t, jnp.sum(a), lvec)
                w = jnp.where(live, jnp.exp(lvec * D_INV_SQRT), 0.0)
                wev = jnp.where(live, w * av, 0.0)

                for t in range(16):
                    ws = w[t]
                    dl = dv[t]
                    for i in range(HID // 16):
                        plsc.addupdate(
                            acc_t.at[dl, pl.ds(16 * i, 16)],
                            qkvb[32 + t, pl.ds(16 * i, 16)] * ws)
                    plsc.addupdate(dacc_t.at[dl], z16f + ws)
                    plsc.addupdate(sacc_t.at[dl], z16f + wev[t])
                return carry3

            lax.fori_loop(0, nblk, blk, 0)

        pltpu.async_copy(pk_h.at[0], pb0, semA)

        def body(j, carry2):
            s0 = 2 * j
            pltpu.async_copy(pk_h.at[s0 + 1], pb1, semB)
            pltpu.make_async_copy(pk_h.at[0], pb0, semA).wait()
            process(s0, pb0)

            @pl.when(s0 + 2 < NSTRIPS)
            def _():
                pltpu.async_copy(pk_h.at[s0 + 2], pb0, semA)

            pltpu.make_async_copy(pk_h.at[0], pb1, semB).wait()
            process(s0 + 1, pb1)
            return carry2

        lax.fori_loop(0, NSTRIPS // 2, body, 0)
        pltpu.sync_copy(acc_t, out_h.at[pl.ds(lo, RTILE)])
        pltpu.sync_copy(dacc_t, den_h.at[pl.ds(lo, RTILE)])
        pltpu.sync_copy(sacc_t, s2_h.at[pl.ds(lo, RTILE)])
        return carry

    lax.fori_loop(0, PASSES, do_pass, 0)


def _edge_phase(q, k, v, we_row, src, dst, ea):
    mesh = plsc.VectorSubcoreMesh(core_axis_name="c", subcore_axis_name="s")
    f32, i32 = jnp.float32, jnp.int32
    ek = functools.partial(
        pl.kernel,
        out_type=[
            jax.ShapeDtypeStruct((NP, HID), f32),
            jax.ShapeDtypeStruct((NP, 16), f32),
            jax.ShapeDtypeStruct((NP, 16), f32),
        ],
        mesh=mesh,
        compiler_params=pltpu.CompilerParams(needs_layout_passes=False),
        scratch_types=[
            pltpu.VMEM((HID,), f32),
            pltpu.VMEM((3, STRIP), i32),
            pltpu.VMEM((3, STRIP), i32),
            pltpu.VMEM((MBUF,), i32),
            pltpu.VMEM((MBUF,), i32),
            pltpu.VMEM((MBUF,), f32),
            pltpu.VMEM((3 * G,), i32),
            pltpu.VMEM((3 * G, HID), f32),
            pltpu.VMEM((RTILE, HID), f32),
            pltpu.VMEM((RTILE, 16), f32),
            pltpu.VMEM((RTILE, 16), f32),
            pltpu.SemaphoreType.DMA,
            pltpu.SemaphoreType.DMA,
            pltpu.SemaphoreType.DMA,
        ],
    )(_edge_kernel)
    qkv = jnp.concatenate([q, k, v], axis=0)
    packed = jnp.stack([
        dst.reshape(NSTRIPS, STRIP),
        src.reshape(NSTRIPS, STRIP),
        lax.bitcast_convert_type(ea, jnp.int32).reshape(NSTRIPS, STRIP),
    ], axis=1)
    out_p, den_p, s2_p = ek(qkv, we_row, packed)
    return out_p[:N_NODES], den_p[:N_NODES, 0], s2_p[:N_NODES, 0]


def _gather_kernel(t_h, i_h, o_h, idx_v, rows_v, sem):
    wid = lax.axis_index("s") * 2 + lax.axis_index("c")
    base = wid * 512
    for ch in range(4):
        off = base + ch * 128
        pltpu.sync_copy(i_h.at[pl.ds(off, 128)], idx_v)
        pltpu.async_copy(t_h.at[idx_v], rows_v, sem).wait()
        pltpu.sync_copy(rows_v, o_h.at[pl.ds(off, 128)])


def _gather_rows(table, idx):
    mesh = plsc.VectorSubcoreMesh(core_axis_name="c", subcore_axis_name="s")
    gk = functools.partial(
        pl.kernel,
        out_type=jax.ShapeDtypeStruct((idx.shape[0], HID), jnp.float32),
        mesh=mesh,
        compiler_params=pltpu.CompilerParams(needs_layout_passes=False),
        scratch_types=[
            pltpu.VMEM((128,), jnp.int32),
            pltpu.VMEM((128, HID), jnp.float32),
            pltpu.SemaphoreType.DMA,
        ],
    )(_gather_kernel)
    return gk(table, idx)


def _layer(h, src, dst, ea, p, pad_k):
    n = h.shape[0]
    din = h.shape[1]
    if pad_k:
        hp = jnp.pad(h, ((0, 0), (0, pad_k)))
        wcat = jnp.pad(
            jnp.concatenate([p["Wq"], p["Wk"], p["Wv"], p["Wskip"]], axis=1),
            ((0, pad_k), (0, 0)))
    else:
        hp = h
        wcat = jnp.concatenate([p["Wq"], p["Wk"], p["Wv"], p["Wskip"]], axis=1)
    bcat = jnp.concatenate([p["bq"], p["bk"], p["bv"], p["bskip"]])
    qkvr = _mm_bias(hp, wcat, bcat)
    q = qkvr[:, :HID]
    k = qkvr[:, HID:2 * HID]
    v = qkvr[:, 2 * HID:3 * HID]
    r = qkvr[:, 3 * HID:]

    we_row = p["We"][0]
    out_raw, denom, s2 = _edge_phase(q, k, v, we_row, src, dst, ea)

    wb = p["Wbeta"][:, 0]
    w_ac = wb[:HID] + wb[2 * HID:]
    w_bc = wb[HID:2 * HID] - wb[2 * HID:]
    return _gate(out_raw, r, denom, s2, we_row, w_ac, w_bc)


def kernel(x, edge_index, edge_attr, start_idx, end_idx, x_1, x_2, params):
    src = edge_index[0]
    dst = edge_index[1]
    ea = edge_attr[:, 0]

    h = _layer(x, src, dst, ea, params["conv0"], pad_k=26)
    h = _layer(h, src, dst, ea, params["conv1"], pad_k=0)
    h = _layer(h, src, dst, ea, params["conv2"], pad_k=0)

    idx = jnp.stack([start_idx, end_idx, x_1, x_2], axis=1).reshape(-1)
    rows = _gather_rows(h, idx.astype(jnp.int32))
    feats = rows.reshape(start_idx.shape[0], 4 * HID)
    return _mlp(feats, params["mlp"])


# speedup vs baseline: 1.0287x; 1.0027x over previous
import functools

import jax
import jax.numpy as jnp
from jax import lax
from jax.experimental import pallas as pl
from jax.experimental.pallas import tpu as pltpu
from jax.experimental.pallas import tpu_sc as plsc

N_NODES = 10000
N_EDGES = 160000
HID = 512
D_INV_SQRT = 1.0 / (512.0 ** 0.5)
EPS = 1e-16

BM = 400

NP = 10240
RTILE = 80
PASSES = NP // (32 * RTILE)
STRIP = 1600
MBUF = 1600
G = 16
NSTRIPS = N_EDGES // STRIP


def _mm_bias_body(x_ref, w_ref, b_ref, o_ref):
    o_ref[...] = (
        jnp.dot(x_ref[...], w_ref[...], preferred_element_type=jnp.float32)
        + b_ref[...]
    )


def _mm_bias(x, w, b, bm=BM):
    m, kdim = x.shape
    n = w.shape[1]
    return pl.pallas_call(
        _mm_bias_body,
        grid=(m // bm,),
        in_specs=[
            pl.BlockSpec((bm, kdim), lambda i: (i, 0)),
            pl.BlockSpec((kdim, n), lambda i: (0, 0)),
            pl.BlockSpec((1, n), lambda i: (0, 0)),
        ],
        out_specs=pl.BlockSpec((bm, n), lambda i: (i, 0)),
        out_shape=jax.ShapeDtypeStruct((m, n), jnp.float32),
    )(x, w, b.reshape(1, n))


def _gate_body(raw_ref, r_ref, d_ref, s2_ref, we_ref, ac_ref, bc_ref, o_ref):
    inv_d = 1.0 / (d_ref[...] + EPS)
    out = (raw_ref[...] + s2_ref[...] * we_ref[...]) * inv_d
    r = r_ref[...]
    bl = jnp.sum(out * ac_ref[...] + r * bc_ref[...], axis=1, keepdims=True)
    beta = jax.nn.sigmoid(bl)
    o_ref[...] = jnp.maximum(beta * r + (1.0 - beta) * out, 0.0)


def _gate(out_raw, r, denom, s2, we_row, w_ac, w_bc):
    n = out_raw.shape[0]
    vspec = pl.BlockSpec((1, HID), lambda i: (0, 0))
    return pl.pallas_call(
        _gate_body,
        grid=(n // BM,),
        in_specs=[
            pl.BlockSpec((BM, HID), lambda i: (i, 0)),
            pl.BlockSpec((BM, HID), lambda i: (i, 0)),
            pl.BlockSpec((BM, 1), lambda i: (i, 0)),
            pl.BlockSpec((BM, 1), lambda i: (i, 0)),
            vspec, vspec, vspec,
        ],
        out_specs=pl.BlockSpec((BM, HID), lambda i: (i, 0)),
        out_shape=jax.ShapeDtypeStruct((n, HID), jnp.float32),
    )(out_raw, r, denom.reshape(n, 1), s2.reshape(n, 1),
      we_row.reshape(1, HID), w_ac.reshape(1, HID), w_bc.reshape(1, HID))


def _mlp_body(f_ref, w1_ref, b1_ref, g_ref, be_ref, w2_ref, b2_ref,
              wh_ref, bh_ref, o_ref):
    h = jnp.dot(f_ref[...], w1_ref[...], preferred_element_type=jnp.float32)
    h = jnp.maximum(h + b1_ref[...], 0.0)
    mu = jnp.mean(h, axis=1, keepdims=True)
    var = jnp.mean((h - mu) ** 2, axis=1, keepdims=True)
    h = (h - mu) * jax.lax.rsqrt(var + 1e-5) * g_ref[...] + be_ref[...]
    h = jnp.dot(h, w2_ref[...], preferred_element_type=jnp.float32)
    h = jnp.maximum(h + b2_ref[...], 0.0)
    o_ref[...] = (
        jnp.dot(h, wh_ref[...], preferred_element_type=jnp.float32)
        + bh_ref[...]
    )


def _mlp(feats, mp):
    b = feats.shape[0]
    bm = 512
    d1 = mp["W1"].shape[0]
    d2 = mp["W1"].shape[1]
    d3 = mp["W2"].shape[1]
    return pl.pallas_call(
        _mlp_body,
        grid=(b // bm,),
        in_specs=[
            pl.BlockSpec((bm, d1), lambda i: (i, 0)),
            pl.BlockSpec((d1, d2), lambda i: (0, 0)),
            pl.BlockSpec((1, d2), lambda i: (0, 0)),
            pl.BlockSpec((1, d2), lambda i: (0, 0)),
            pl.BlockSpec((1, d2), lambda i: (0, 0)),
            pl.BlockSpec((d2, d3), lambda i: (0, 0)),
            pl.BlockSpec((1, d3), lambda i: (0, 0)),
            pl.BlockSpec((d3, 1), lambda i: (0, 0)),
            pl.BlockSpec((1, 1), lambda i: (0, 0)),
        ],
        out_specs=pl.BlockSpec((bm, 1), lambda i: (i, 0)),
        out_shape=jax.ShapeDtypeStruct((b, 1), jnp.float32),
    )(feats, mp["W1"], mp["b1"].reshape(1, d2), mp["ln_g"].reshape(1, d2),
      mp["ln_b"].reshape(1, d2), mp["W2"], mp["b2"].reshape(1, d3),
      mp["Wh"], mp["bh"].reshape(1, 1))


def _edge_kernel(qkv_h, we_h, pk_h,
                 out_h, den_h, s2_h,
                 we_v, pb0, pb1, srcm, dlm, eam, qkvi, qkvb,
                 acc_t, dacc_t, sacc_t, semA, semB, semG):
    c = lax.axis_index("c")
    s = lax.axis_index("s")
    wid = s * 2 + c
    z16f = jnp.zeros((16,), jnp.float32)
    lane = lax.iota(jnp.int32, 16)
    pltpu.sync_copy(we_h, we_v)

    def do_pass(p, carry):
        lo = p * (32 * RTILE) + wid * RTILE

        def zrow(rr, carry2):
            for i in range(HID // 16):
                acc_t[rr, pl.ds(16 * i, 16)] = z16f
            dacc_t[rr, :] = z16f
            sacc_t[rr, :] = z16f
            return carry2

        lax.fori_loop(0, RTILE, zrow, 0)

        def process(st, pb):
            def filt(t, off):
                grp = []
                for u in range(4):
                    ofs = 64 * t + 16 * u
                    d16 = pb[0, pl.ds(ofs, 16)]
                    m = (d16 >= lo) & (d16 < lo + RTILE)
                    cm = plsc.all_reduce_population_count(m)[0]
                    grp.append((d16, m, cm, ofs))
                o = off
                for d16, m, cm, ofs in grp:
                    plsc.store_compressed(srcm.at[pl.ds(o, 16)],
                                          pb[1, pl.ds(ofs, 16)], mask=m)
                    plsc.store_compressed(dlm.at[pl.ds(o, 16)], d16 - lo,
                                          mask=m)
                    plsc.store_compressed(
                        eam.at[pl.ds(o, 16)],
                        plsc.bitcast(pb[2, pl.ds(ofs, 16)], jnp.float32),
                        mask=m)
                    o = o + cm
                return o

            mcount = lax.fori_loop(0, STRIP // 64, filt, 0)
            nblk = (mcount + G - 1) // G

            def blk(g, carry3):
                gb = g * G
                pos = gb + lane
                live = pos < mcount
                sv = jnp.where(live, srcm[pl.ds(gb, 16)], 0)
                dv = jnp.where(live, dlm[pl.ds(gb, 16)], 0)
                av = eam[pl.ds(gb, 16)]
                qkvi[pl.ds(0, 16)] = jnp.where(live, dv + lo, 0)
                qkvi[pl.ds(16, 16)] = sv + N_NODES
                qkvi[pl.ds(32, 16)] = sv + 2 * N_NODES
                pltpu.async_copy(qkv_h.at[qkvi], qkvb, semG).wait()

                lvec = z16f
                for t in range(16):
                    ea_e = av[t]
                    a0 = z16f
                    a1 = z16f
                    a2 = z16f
                    a3 = z16f
                    for i4 in range(0, HID // 16, 4):
                        prods = []
                        for u in range(4):
                            off_i = 16 * (i4 + u)
                            kk = (qkvb[16 + t, pl.ds(off_i, 16)]
                                  + ea_e * we_v[pl.ds(off_i, 16)])
                            prods.append(qkvb[t, pl.ds(off_i, 16)] * kk)
                        a0 = a0 + prods[0]
                        a1 = a1 + prods[1]
                        a2 = a2 + prods[2]
                        a3 = a3 + prods[3]
                    a = (a0 + a1) + (a2 + a3)
                    lvec = jnp.where(lane == t, jnp.sum(a), lvec)
                w = jnp.where(live, jnp.exp(lvec * D_INV_SQRT), 0.0)
                wev = jnp.where(live, w * av, 0.0)

                for t in range(16):
                    ws = w[t]
                    dl = dv[t]
                    for i in range(HID // 16):
                        plsc.addupdate(
                            acc_t.at[dl, pl.ds(16 * i, 16)],
                            qkvb[32 + t, pl.ds(16 * i, 16)] * ws)
                    plsc.addupdate(dacc_t.at[dl], z16f + ws)
                    plsc.addupdate(sacc_t.at[dl], z16f + wev[t])
                return carry3

            lax.fori_loop(0, nblk, blk, 0)

        pltpu.async_copy(pk_h.at[0], pb0, semA)

        def body(j, carry2):
            s0 = 2 * j
            pltpu.async_copy(pk_h.at[s0 + 1], pb1, semB)
            pltpu.make_async_copy(pk_h.at[0], pb0, semA).wait()
            process(s0, pb0)

            @pl.when(s0 + 2 < NSTRIPS)
            def _():
                pltpu.async_copy(pk_h.at[s0 + 2], pb0, semA)

            pltpu.make_async_copy(pk_h.at[0], pb1, semB).wait()
            process(s0 + 1, pb1)
            return carry2

        lax.fori_loop(0, NSTRIPS // 2, body, 0)
        pltpu.sync_copy(acc_t, out_h.at[pl.ds(lo, RTILE)])
        pltpu.sync_copy(dacc_t, den_h.at[pl.ds(lo, RTILE)])
        pltpu.sync_copy(sacc_t, s2_h.at[pl.ds(lo, RTILE)])
        return carry

    lax.fori_loop(0, PASSES, do_pass, 0)


def _edge_phase(q, k, v, we_row, src, dst, ea):
    mesh = plsc.VectorSubcoreMesh(core_axis_name="c", subcore_axis_name="s")
    f32, i32 = jnp.float32, jnp.int32
    ek = functools.partial(
        pl.kernel,
        out_type=[
            jax.ShapeDtypeStruct((NP, HID), f32),
            jax.ShapeDtypeStruct((NP, 16), f32),
            jax.ShapeDtypeStruct((NP, 16), f32),
        ],
        mesh=mesh,
        compiler_params=pltpu.CompilerParams(needs_layout_passes=False),
        scratch_types=[
            pltpu.VMEM((HID,), f32),
            pltpu.VMEM((3, STRIP), i32),
            pltpu.VMEM((3, STRIP), i32),
            pltpu.VMEM((MBUF,), i32),
            pltpu.VMEM((MBUF,), i32),
            pltpu.VMEM((MBUF,), f32),
            pltpu.VMEM((3 * G,), i32),
            pltpu.VMEM((3 * G, HID), f32),
            pltpu.VMEM((RTILE, HID), f32),
            pltpu.VMEM((RTILE, 16), f32),
            pltpu.VMEM((RTILE, 16), f32),
            pltpu.SemaphoreType.DMA,
            pltpu.SemaphoreType.DMA,
            pltpu.SemaphoreType.DMA,
        ],
    )(_edge_kernel)
    qkv = jnp.concatenate([q, k, v], axis=0)
    packed = jnp.stack([
        dst.reshape(NSTRIPS, STRIP),
        src.reshape(NSTRIPS, STRIP),
        lax.bitcast_convert_type(ea, jnp.int32).reshape(NSTRIPS, STRIP),
    ], axis=1)
    out_p, den_p, s2_p = ek(qkv, we_row, packed)
    return out_p[:N_NODES], den_p[:N_NODES, 0], s2_p[:N_NODES, 0]


def _gather_kernel(t_h, i_h, o_h, idx_v, rows_v, sem):
    wid = lax.axis_index("s") * 2 + lax.axis_index("c")
    base = wid * 512
    for ch in range(4):
        off = base + ch * 128
        pltpu.sync_copy(i_h.at[pl.ds(off, 128)], idx_v)
        pltpu.async_copy(t_h.at[idx_v], rows_v, sem).wait()
        pltpu.sync_copy(rows_v, o_h.at[pl.ds(off, 128)])


def _gather_rows(table, idx):
    mesh = plsc.VectorSubcoreMesh(core_axis_name="c", subcore_axis_name="s")
    gk = functools.partial(
        pl.kernel,
        out_type=jax.ShapeDtypeStruct((idx.shape[0], HID), jnp.float32),
        mesh=mesh,
        compiler_params=pltpu.CompilerParams(needs_layout_passes=False),
        scratch_types=[
            pltpu.VMEM((128,), jnp.int32),
            pltpu.VMEM((128, HID), jnp.float32),
            pltpu.SemaphoreType.DMA,
        ],
    )(_gather_kernel)
    return gk(table, idx)


def _layer(h, src, dst, ea, p, pad_k):
    n = h.shape[0]
    din = h.shape[1]
    if pad_k:
        hp = jnp.pad(h, ((0, 0), (0, pad_k)))
        wcat = jnp.pad(
            jnp.concatenate([p["Wq"], p["Wk"], p["Wv"], p["Wskip"]], axis=1),
            ((0, pad_k), (0, 0)))
    else:
        hp = h
        wcat = jnp.concatenate([p["Wq"], p["Wk"], p["Wv"], p["Wskip"]], axis=1)
    bcat = jnp.concatenate([p["bq"], p["bk"], p["bv"], p["bskip"]])
    qkvr = _mm_bias(hp, wcat, bcat)
    q = qkvr[:, :HID]
    k = qkvr[:, HID:2 * HID]
    v = qkvr[:, 2 * HID:3 * HID]
    r = qkvr[:, 3 * HID:]

    we_row = p["We"][0]
    out_raw, denom, s2 = _edge_phase(q, k, v, we_row, src, dst, ea)

    wb = p["Wbeta"][:, 0]
    w_ac = wb[:HID] + wb[2 * HID:]
    w_bc = wb[HID:2 * HID] - wb[2 * HID:]
    return _gate(out_raw, r, denom, s2, we_row, w_ac, w_bc)


def kernel(x, edge_index, edge_attr, start_idx, end_idx, x_1, x_2, params):
    src = edge_index[0]
    dst = edge_index[1]
    ea = edge_attr[:, 0]

    h = _layer(x, src, dst, ea, params["conv0"], pad_k=26)
    h = _layer(h, src, dst, ea, params["conv1"], pad_k=0)
    h = _layer(h, src, dst, ea, params["conv2"], pad_k=0)

    idx = jnp.stack([start_idx, end_idx, x_1, x_2], axis=1).reshape(-1)
    rows = _gather_rows(h, idx.astype(jnp.int32))
    feats = rows.reshape(start_idx.shape[0], 4 * HID)
    return _mlp(feats, params["mlp"])


# speedup vs baseline: 1.0294x; 1.0007x over previous
import functools

import jax
import jax.numpy as jnp
from jax import lax
from jax.experimental import pallas as pl
from jax.experimental.pallas import tpu as pltpu
from jax.experimental.pallas import tpu_sc as plsc

N_NODES = 10000
N_EDGES = 160000
HID = 512
D_INV_SQRT = 1.0 / (512.0 ** 0.5)
EPS = 1e-16

BM = 400

RTILE = 80
PASSES = 4
NP = PASSES * 32 * RTILE
STRIP = 1600
MBUF = 1600
G = 16
NSTRIPS = N_EDGES // STRIP


def _mm_bias_body(x_ref, w_ref, b_ref, o_ref):
    o_ref[...] = (
        jnp.dot(x_ref[...], w_ref[...], preferred_element_type=jnp.float32)
        + b_ref[...]
    )


def _mm_bias(x, w, b, bm=BM):
    m, kdim = x.shape
    n = w.shape[1]
    return pl.pallas_call(
        _mm_bias_body,
        grid=(m // bm,),
        in_specs=[
            pl.BlockSpec((bm, kdim), lambda i: (i, 0)),
            pl.BlockSpec((kdim, n), lambda i: (0, 0)),
            pl.BlockSpec((1, n), lambda i: (0, 0)),
        ],
        out_specs=pl.BlockSpec((bm, n), lambda i: (i, 0)),
        out_shape=jax.ShapeDtypeStruct((m, n), jnp.float32),
    )(x, w, b.reshape(1, n))


def _gate_body(raw_ref, r_ref, d_ref, s2_ref, we_ref, ac_ref, bc_ref, o_ref):
    inv_d = 1.0 / (d_ref[...] + EPS)
    out = (raw_ref[...] + s2_ref[...] * we_ref[...]) * inv_d
    r = r_ref[...]
    bl = jnp.sum(out * ac_ref[...] + r * bc_ref[...], axis=1, keepdims=True)
    beta = jax.nn.sigmoid(bl)
    o_ref[...] = jnp.maximum(beta * r + (1.0 - beta) * out, 0.0)


def _gate(out_raw, r, denom, s2, we_row, w_ac, w_bc):
    n = out_raw.shape[0]
    vspec = pl.BlockSpec((1, HID), lambda i: (0, 0))
    return pl.pallas_call(
        _gate_body,
        grid=(n // BM,),
        in_specs=[
            pl.BlockSpec((BM, HID), lambda i: (i, 0)),
            pl.BlockSpec((BM, HID), lambda i: (i, 0)),
            pl.BlockSpec((BM, 1), lambda i: (i, 0)),
            pl.BlockSpec((BM, 1), lambda i: (i, 0)),
            vspec, vspec, vspec,
        ],
        out_specs=pl.BlockSpec((BM, HID), lambda i: (i, 0)),
        out_shape=jax.ShapeDtypeStruct((n, HID), jnp.float32),
    )(out_raw, r, denom.reshape(n, 1), s2.reshape(n, 1),
      we_row.reshape(1, HID), w_ac.reshape(1, HID), w_bc.reshape(1, HID))


def _mlp_body(f_ref, w1_ref, b1_ref, g_ref, be_ref, w2_ref, b2_ref,
              wh_ref, bh_ref, o_ref):
    h = jnp.dot(f_ref[...], w1_ref[...], preferred_element_type=jnp.float32)
    h = jnp.maximum(h + b1_ref[...], 0.0)
    mu = jnp.mean(h, axis=1, keepdims=True)
    var = jnp.mean((h - mu) ** 2, axis=1, keepdims=True)
    h = (h - mu) * jax.lax.rsqrt(var + 1e-5) * g_ref[...] + be_ref[...]
    h = jnp.dot(h, w2_ref[...], preferred_element_type=jnp.float32)
    h = jnp.maximum(h + b2_ref[...], 0.0)
    o_ref[...] = (
        jnp.dot(h, wh_ref[...], preferred_element_type=jnp.float32)
        + bh_ref[...]
    )


def _mlp(feats, mp):
    b = feats.shape[0]
    bm = 512
    d1 = mp["W1"].shape[0]
    d2 = mp["W1"].shape[1]
    d3 = mp["W2"].shape[1]
    return pl.pallas_call(
        _mlp_body,
        grid=(b // bm,),
        in_specs=[
            pl.BlockSpec((bm, d1), lambda i: (i, 0)),
            pl.BlockSpec((d1, d2), lambda i: (0, 0)),
            pl.BlockSpec((1, d2), lambda i: (0, 0)),
            pl.BlockSpec((1, d2), lambda i: (0, 0)),
            pl.BlockSpec((1, d2), lambda i: (0, 0)),
            pl.BlockSpec((d2, d3), lambda i: (0, 0)),
            pl.BlockSpec((1, d3), lambda i: (0, 0)),
            pl.BlockSpec((d3, 1), lambda i: (0, 0)),
            pl.BlockSpec((1, 1), lambda i: (0, 0)),
        ],
        out_specs=pl.BlockSpec((bm, 1), lambda i: (i, 0)),
        out_shape=jax.ShapeDtypeStruct((b, 1), jnp.float32),
    )(feats, mp["W1"], mp["b1"].reshape(1, d2), mp["ln_g"].reshape(1, d2),
      mp["ln_b"].reshape(1, d2), mp["W2"], mp["b2"].reshape(1, d3),
      mp["Wh"], mp["bh"].reshape(1, 1))


def _edge_kernel(qkv_h, we_h, pk_h,
                 out_h, den_h, s2_h,
                 we_v, pb0, pb1, srcm, dlm, eam, qkvi, qkvb,
                 acc_t, dacc_t, sacc_t, semA, semB, semG):
    c = lax.axis_index("c")
    s = lax.axis_index("s")
    wid = s * 2 + c
    z16f = jnp.zeros((16,), jnp.float32)
    lane = lax.iota(jnp.int32, 16)
    pltpu.sync_copy(we_h, we_v)

    def do_pass(p, carry):
        lo = p * (32 * RTILE) + wid * RTILE

        def zrow(rr, carry2):
            for i in range(HID // 16):
                acc_t[rr, pl.ds(16 * i, 16)] = z16f
            dacc_t[rr, :] = z16f
            sacc_t[rr, :] = z16f
            return carry2

        lax.fori_loop(0, RTILE, zrow, 0)

        def process(st, pb):
            def filt(t, off):
                grp = []
                for u in range(4):
                    ofs = 64 * t + 16 * u
                    d16 = pb[0, pl.ds(ofs, 16)]
                    m = (d16 >= lo) & (d16 < lo + RTILE)
                    cm = plsc.all_reduce_population_count(m)[0]
                    grp.append((d16, m, cm, ofs))
                o = off
                for d16, m, cm, ofs in grp:
                    plsc.store_compressed(srcm.at[pl.ds(o, 16)],
                                          pb[1, pl.ds(ofs, 16)], mask=m)
                    plsc.store_compressed(dlm.at[pl.ds(o, 16)], d16 - lo,
                                          mask=m)
                    plsc.store_compressed(
                        eam.at[pl.ds(o, 16)],
                        plsc.bitcast(pb[2, pl.ds(ofs, 16)], jnp.float32),
                        mask=m)
                    o = o + cm
                return o

            mcount = lax.fori_loop(0, STRIP // 64, filt, 0)
            nblk = (mcount + G - 1) // G

            def blk(g, carry3):
                gb = g * G
                pos = gb + lane
                live = pos < mcount
                sv = jnp.where(live, srcm[pl.ds(gb, 16)], 0)
                dv = jnp.where(live, dlm[pl.ds(gb, 16)], 0)
                av = eam[pl.ds(gb, 16)]
                qkvi[pl.ds(0, 16)] = jnp.where(live, dv + lo, 0)
                qkvi[pl.ds(16, 16)] = sv + N_NODES
                qkvi[pl.ds(32, 16)] = sv + 2 * N_NODES
                pltpu.async_copy(qkv_h.at[qkvi], qkvb, semG).wait()

                lvec = z16f
                for t in range(16):
                    ea_e = av[t]
                    a0 = z16f
                    a1 = z16f
                    a2 = z16f
                    a3 = z16f
                    for i4 in range(0, HID // 16, 4):
                        prods = []
                        for u in range(4):
                            off_i = 16 * (i4 + u)
                            kk = (qkvb[16 + t, pl.ds(off_i, 16)]
                                  + ea_e * we_v[pl.ds(off_i, 16)])
                            prods.append(qkvb[t, pl.ds(off_i, 16)] * kk)
                        a0 = a0 + prods[0]
                        a1 = a1 + prods[1]
                        a2 = a2 + prods[2]
                        a3 = a3 + prods[3]
                    a = (a0 + a1) + (a2 + a3)
                    lvec = jnp.where(lane == t, jnp.sum(a), lvec)
                w = jnp.where(live, jnp.exp(lvec * D_INV_SQRT), 0.0)
                wev = jnp.where(live, w * av, 0.0)

                for t in range(16):
                    ws = w[t]
                    dl = dv[t]
                    for i in range(HID // 16):
                        plsc.addupdate(
                            acc_t.at[dl, pl.ds(16 * i, 16)],
                            qkvb[32 + t, pl.ds(16 * i, 16)] * ws)
                    plsc.addupdate(dacc_t.at[dl], z16f + ws)
                    plsc.addupdate(sacc_t.at[dl], z16f + wev[t])
                return carry3

            lax.fori_loop(0, nblk, blk, 0)

        pltpu.async_copy(pk_h.at[0], pb0, semA)

        def body(j, carry2):
            s0 = 2 * j
            pltpu.async_copy(pk_h.at[s0 + 1], pb1, semB)
            pltpu.make_async_copy(pk_h.at[0], pb0, semA).wait()
            process(s0, pb0)

            @pl.when(s0 + 2 < NSTRIPS)
            def _():
                pltpu.async_copy(pk_h.at[s0 + 2], pb0, semA)

            pltpu.make_async_copy(pk_h.at[0], pb1, semB).wait()
            process(s0 + 1, pb1)
            return carry2

        lax.fori_loop(0, NSTRIPS // 2, body, 0)
        pltpu.sync_copy(acc_t, out_h.at[pl.ds(lo, RTILE)])
        pltpu.sync_copy(dacc_t, den_h.at[pl.ds(lo, RTILE)])
        pltpu.sync_copy(sacc_t, s2_h.at[pl.ds(lo, RTILE)])
        return carry

    lax.fori_loop(0, PASSES, do_pass, 0)


def _edge_phase(q, k, v, we_row, src, dst, ea):
    mesh = plsc.VectorSubcoreMesh(core_axis_name="c", subcore_axis_name="s")
    f32, i32 = jnp.float32, jnp.int32
    ek = functools.partial(
        pl.kernel,
        out_type=[
            jax.ShapeDtypeStruct((NP, HID), f32),
            jax.ShapeDtypeStruct((NP, 16), f32),
            jax.ShapeDtypeStruct((NP, 16), f32),
        ],
        mesh=mesh,
        compiler_params=pltpu.CompilerParams(needs_layout_passes=False),
        scratch_types=[
            pltpu.VMEM((HID,), f32),
            pltpu.VMEM((3, STRIP), i32),
            pltpu.VMEM((3, STRIP), i32),
            pltpu.VMEM((MBUF,), i32),
            pltpu.VMEM((MBUF,), i32),
            pltpu.VMEM((MBUF,), f32),
            pltpu.VMEM((3 * G,), i32),
            pltpu.VMEM((3 * G, HID), f32),
            pltpu.VMEM((RTILE, HID), f32),
            pltpu.VMEM((RTILE, 16), f32),
            pltpu.VMEM((RTILE, 16), f32),
            pltpu.SemaphoreType.DMA,
            pltpu.SemaphoreType.DMA,
            pltpu.SemaphoreType.DMA,
        ],
    )(_edge_kernel)
    qkv = jnp.concatenate([q, k, v], axis=0)
    packed = jnp.stack([
        dst.reshape(NSTRIPS, STRIP),
        src.reshape(NSTRIPS, STRIP),
        lax.bitcast_convert_type(ea, jnp.int32).reshape(NSTRIPS, STRIP),
    ], axis=1)
    out_p, den_p, s2_p = ek(qkv, we_row, packed)
    return out_p[:N_NODES], den_p[:N_NODES, 0], s2_p[:N_NODES, 0]


def _gather_kernel(t_h, i_h, o_h, idx_v, rows_v, sem):
    wid = lax.axis_index("s") * 2 + lax.axis_index("c")
    base = wid * 512
    for ch in range(4):
        off = base + ch * 128
        pltpu.sync_copy(i_h.at[pl.ds(off, 128)], idx_v)
        pltpu.async_copy(t_h.at[idx_v], rows_v, sem).wait()
        pltpu.sync_copy(rows_v, o_h.at[pl.ds(off, 128)])


def _gather_rows(table, idx):
    mesh = plsc.VectorSubcoreMesh(core_axis_name="c", subcore_axis_name="s")
    gk = functools.partial(
        pl.kernel,
        out_type=jax.ShapeDtypeStruct((idx.shape[0], HID), jnp.float32),
        mesh=mesh,
        compiler_params=pltpu.CompilerParams(needs_layout_passes=False),
        scratch_types=[
            pltpu.VMEM((128,), jnp.int32),
            pltpu.VMEM((128, HID), jnp.float32),
            pltpu.SemaphoreType.DMA,
        ],
    )(_gather_kernel)
    return gk(table, idx)


def _layer(h, src, dst, ea, p, pad_k):
    n = h.shape[0]
    din = h.shape[1]
    if pad_k:
        hp = jnp.pad(h, ((0, 0), (0, pad_k)))
        wcat = jnp.pad(
            jnp.concatenate([p["Wq"], p["Wk"], p["Wv"], p["Wskip"]], axis=1),
            ((0, pad_k), (0, 0)))
    else:
        hp = h
        wcat = jnp.concatenate([p["Wq"], p["Wk"], p["Wv"], p["Wskip"]], axis=1)
    bcat = jnp.concatenate([p["bq"], p["bk"], p["bv"], p["bskip"]])
    qkvr = _mm_bias(hp, wcat, bcat)
    q = qkvr[:, :HID]
    k = qkvr[:, HID:2 * HID]
    v = qkvr[:, 2 * HID:3 * HID]
    r = qkvr[:, 3 * HID:]

    we_row = p["We"][0]
    out_raw, denom, s2 = _edge_phase(q, k, v, we_row, src, dst, ea)

    wb = p["Wbeta"][:, 0]
    w_ac = wb[:HID] + wb[2 * HID:]
    w_bc = wb[HID:2 * HID] - wb[2 * HID:]
    return _gate(out_raw, r, denom, s2, we_row, w_ac, w_bc)


def kernel(x, edge_index, edge_attr, start_idx, end_idx, x_1, x_2, params):
    src = edge_index[0]
    dst = edge_index[1]
    ea = edge_attr[:, 0]

    h = _layer(x, src, dst, ea, params["conv0"], pad_k=26)
    h = _layer(h, src, dst, ea, params["conv1"], pad_k=0)
    h = _layer(h, src, dst, ea, params["conv2"], pad_k=0)

    idx = jnp.stack([start_idx, end_idx, x_1, x_2], axis=1).reshape(-1)
    rows = _gather_rows(h, idx.astype(jnp.int32))
    feats = rows.reshape(start_idx.shape[0], 4 * HID)
    return _mlp(feats, params["mlp"])
